# baseline (device time: 156549 ns/iter reference)
import jax
import jax.numpy as jnp
from jax import lax
from jax.experimental import pallas as pl
from jax.experimental.pallas import tpu as pltpu

N_Y = 4
N_Z = 4
N_X = 2
K = 4


def kernel(x):
    _, m, n_tot = x.shape
    chunk = n_tot // N_Y
    strip = m // (N_X * N_Z)
    sub = strip // K

    def body(
        x_hbm, out_ref, comm_ref, stage_ref,
        copy_sems, p1_send, p1_recv,
        z_send, z_recv, x_send, x_recv,
    ):
        my_x = lax.axis_index("x")
        my_y = lax.axis_index("y")
        my_z = lax.axis_index("z")
        right = (my_y + 1) % N_Y
        left = (my_y + N_Y - 1) % N_Y

        row0 = (N_Z * my_x + my_z) * strip

        barrier = pltpu.get_barrier_semaphore()
        pl.semaphore_signal(barrier, inc=1, device_id=(my_x, left, my_z),
                            device_id_type=pl.DeviceIdType.MESH)
        pl.semaphore_signal(barrier, inc=1, device_id=(my_x, right, my_z),
                            device_id_type=pl.DeviceIdType.MESH)
        pl.semaphore_signal(barrier, inc=1, device_id=(1 - my_x, my_y, my_z),
                            device_id_type=pl.DeviceIdType.MESH)
        for zo in range(1, N_Z):
            pl.semaphore_signal(barrier, inc=1,
                                device_id=(my_x, my_y, (my_z + zo) % N_Z),
                                device_id_type=pl.DeviceIdType.MESH)
        pl.semaphore_wait(barrier, 6)

        def addend(s):
            return (my_y + N_Y - 1 - s) % N_Y

        def stage_copy(k, c):
            return pltpu.make_async_copy(
                x_hbm.at[0, pl.ds(row0 + k * sub, sub), pl.ds(c * chunk, chunk)],
                stage_ref.at[k],
                copy_sems.at[k],
            )

        def hop_desc(k, s):
            if s == 0:
                src = x_hbm.at[0, pl.ds(row0 + k * sub, sub),
                               pl.ds(addend(0) * chunk, chunk)]
            else:
                src = comm_ref.at[k, s - 1]
            return pltpu.make_async_remote_copy(
                src_ref=src,
                dst_ref=comm_ref.at[k, s],
                send_sem=p1_send.at[k, s],
                recv_sem=p1_recv.at[k, s],
                device_id=(my_x, right, my_z),
                device_id_type=pl.DeviceIdType.MESH,
            )

        def p1_start(k, s):
            if s > 0:
                stage_copy(k, addend(s)).wait()
                comm_ref[k, s - 1] += stage_ref[k]
            hop_desc(k, s).start()
            stage_copy(k, addend(s + 1)).start()

        def p1_wait(k, s):
            hop_desc(k, s).wait_recv()

        def piece_rows(o, k, xside=None):
            xi = my_x if xside is None else xside
            return pl.ds((N_Z * xi + o) * strip + k * sub, sub)

        def z_desc(k, origin, target):
            return pltpu.make_async_remote_copy(
                src_ref=out_ref.at[piece_rows(origin, k), :],
                dst_ref=out_ref.at[piece_rows(origin, k), :],
                send_sem=z_send.at[k, target],
                recv_sem=z_recv.at[k, origin],
                device_id=(my_x, my_y, target),
                device_id_type=pl.DeviceIdType.MESH,
            )

        def x_push(k, origin):
            return pltpu.make_async_remote_copy(
                src_ref=out_ref.at[piece_rows(origin, k), :],
                dst_ref=out_ref.at[piece_rows(origin, k), :],
                send_sem=x_send.at[k, origin],
                recv_sem=x_recv.at[k, origin],
                device_id=(1 - my_x, my_y, my_z),
                device_id_type=pl.DeviceIdType.MESH,
            )

        def x_recv_desc(k, origin):
            rows = piece_rows(origin, k, xside=1 - my_x)
            return pltpu.make_async_remote_copy(
                src_ref=out_ref.at[rows, :],
                dst_ref=out_ref.at[rows, :],
                send_sem=x_send.at[k, origin],
                recv_sem=x_recv.at[k, origin],
                device_id=(1 - my_x, my_y, my_z),
                device_id_type=pl.DeviceIdType.MESH,
            )

        def p1_finish(k):
            stage_copy(k, my_y).wait()
            out_ref[pl.ds(row0 + k * sub, sub), :] = comm_ref[k, N_Y - 2] + stage_ref[k]
            x_push(k, my_z).start()
            for dz in range(1, N_Z):
                @pl.when(my_z + dz <= N_Z - 1)
                def _():
                    z_desc(k, my_z, my_z + dz).start()

                @pl.when(my_z - dz >= 0)
                def _():
                    z_desc(k, my_z, my_z - dz).start()

        def z_arrival(k, d):
            @pl.when(my_z - d >= 0)
            def _():
                z_desc(k, my_z - d, my_z).wait_recv()
                x_push(k, my_z - d).start()

            @pl.when(my_z + d <= N_Z - 1)
            def _():
                z_desc(k, my_z + d, my_z).wait_recv()
                x_push(k, my_z + d).start()

        p1_start(0, 0)
        for k in range(K):
            p1_wait(k, 0)
            p1_start(k, 1)
            p1_wait(k, 1)
            p1_start(k, 2)
            if k + 1 < K:
                p1_start(k + 1, 0)
            p1_wait(k, 2)
            p1_finish(k)
            if k > 0:
                for d in (1, 2, 3):
                    z_arrival(k - 1, d)
        for d in (1, 2, 3):
            z_arrival(K - 1, d)

        for k in range(K):
            for o in range(N_Z):
                x_recv_desc(k, o).wait_recv()
                x_push(k, o).wait_send()
        for k in range(K):
            for s in range(N_Y - 1):
                hop_desc(k, s).wait_send()
            for dz in range(1, N_Z):
                @pl.when(my_z + dz <= N_Z - 1)
                def _():
                    z_desc(k, my_z, my_z + dz).wait_send()

                @pl.when(my_z - dz >= 0)
                def _():
                    z_desc(k, my_z, my_z - dz).wait_send()

    return pl.pallas_call(
        body,
        out_shape=jax.ShapeDtypeStruct((m, chunk), jnp.float32),
        in_specs=[pl.BlockSpec(memory_space=pl.ANY)],
        out_specs=pl.BlockSpec(memory_space=pltpu.MemorySpace.VMEM),
        scratch_shapes=[
            pltpu.VMEM((K, N_Y - 1, sub, chunk), jnp.float32),
            pltpu.VMEM((K, sub, chunk), jnp.float32),
            pltpu.SemaphoreType.DMA((K,)),
            pltpu.SemaphoreType.DMA((K, N_Y - 1)),
            pltpu.SemaphoreType.DMA((K, N_Y - 1)),
            pltpu.SemaphoreType.DMA((K, N_Z)),
            pltpu.SemaphoreType.DMA((K, N_Z)),
            pltpu.SemaphoreType.DMA((K, N_Z)),
            pltpu.SemaphoreType.DMA((K, N_Z)),
        ],
        compiler_params=pltpu.CompilerParams(
            collective_id=0,
            vmem_limit_bytes=63 * 1024 * 1024,
        ),
    )(x)


# device time: 147366 ns/iter; 1.0623x vs baseline; 1.0623x over previous
import jax
import jax.numpy as jnp
from jax import lax
from jax.experimental import pallas as pl
from jax.experimental.pallas import tpu as pltpu

N_Y = 4
N_Z = 4
N_X = 2
K = 4


def kernel(x):
    _, m, n_tot = x.shape
    chunk = n_tot // N_Y
    strip = m // (N_X * N_Z)
    sub = strip // K

    def body(
        x_hbm, out_ref, comm_ref, stage_ref,
        copy_sems, p1_send, p1_recv,
        ze_send, ze_recv, zw_send, zw_recv,
        x_send, x_recv,
    ):
        my_x = lax.axis_index("x")
        my_y = lax.axis_index("y")
        my_z = lax.axis_index("z")
        right = (my_y + 1) % N_Y
        left = (my_y + N_Y - 1) % N_Y

        row0 = (N_Z * my_x + my_z) * strip

        barrier = pltpu.get_barrier_semaphore()
        pl.semaphore_signal(barrier, inc=1, device_id=(my_x, left, my_z),
                            device_id_type=pl.DeviceIdType.MESH)
        pl.semaphore_signal(barrier, inc=1, device_id=(my_x, right, my_z),
                            device_id_type=pl.DeviceIdType.MESH)
        pl.semaphore_signal(barrier, inc=1, device_id=(1 - my_x, my_y, my_z),
                            device_id_type=pl.DeviceIdType.MESH)

        @pl.when(my_z > 0)
        def _():
            pl.semaphore_signal(barrier, inc=1, device_id=(my_x, my_y, my_z - 1),
                                device_id_type=pl.DeviceIdType.MESH)

        @pl.when(my_z < N_Z - 1)
        def _():
            pl.semaphore_signal(barrier, inc=1, device_id=(my_x, my_y, my_z + 1),
                                device_id_type=pl.DeviceIdType.MESH)

        n_nbrs = 3 + (my_z > 0).astype(jnp.int32) + (my_z < N_Z - 1).astype(jnp.int32)
        pl.semaphore_wait(barrier, n_nbrs)

        def addend(s):
            return (my_y + N_Y - 1 - s) % N_Y

        def stage_copy(k, c):
            return pltpu.make_async_copy(
                x_hbm.at[0, pl.ds(row0 + k * sub, sub), pl.ds(c * chunk, chunk)],
                stage_ref.at[k],
                copy_sems.at[k],
            )

        def hop_desc(k, s):
            if s == 0:
                src = x_hbm.at[0, pl.ds(row0 + k * sub, sub),
                               pl.ds(addend(0) * chunk, chunk)]
            else:
                src = comm_ref.at[k, s - 1]
            return pltpu.make_async_remote_copy(
                src_ref=src,
                dst_ref=comm_ref.at[k, s],
                send_sem=p1_send.at[k, s],
                recv_sem=p1_recv.at[k, s],
                device_id=(my_x, right, my_z),
                device_id_type=pl.DeviceIdType.MESH,
            )

        def p1_start(k, s):
            if s > 0:
                stage_copy(k, addend(s)).wait()
                comm_ref[k, s - 1] += stage_ref[k]
            hop_desc(k, s).start()
            stage_copy(k, addend(s + 1)).start()

        def p1_wait(k, s):
            hop_desc(k, s).wait_recv()

        def p1_finish(k):
            stage_copy(k, my_y).wait()
            out_ref[pl.ds(row0 + k * sub, sub), :] = comm_ref[k, N_Y - 2] + stage_ref[k]
            x_push(k, my_z).start()

        def piece_rows(o, k, xside=None):
            xi = my_x if xside is None else xside
            return pl.ds((N_Z * xi + o) * strip + k * sub, sub)

        def z_rdma(k, origin, dz, send_s, recv_s):
            return pltpu.make_async_remote_copy(
                src_ref=out_ref.at[piece_rows(origin, k), :],
                dst_ref=out_ref.at[piece_rows(origin, k), :],
                send_sem=send_s,
                recv_sem=recv_s,
                device_id=(my_x, my_y, my_z + dz),
                device_id_type=pl.DeviceIdType.MESH,
            )

        def x_push(k, origin):
            return pltpu.make_async_remote_copy(
                src_ref=out_ref.at[piece_rows(origin, k), :],
                dst_ref=out_ref.at[piece_rows(origin, k), :],
                send_sem=x_send.at[k, origin],
                recv_sem=x_recv.at[k, origin],
                device_id=(1 - my_x, my_y, my_z),
                device_id_type=pl.DeviceIdType.MESH,
            )

        def x_recv_desc(k, origin):
            rows = piece_rows(origin, k, xside=1 - my_x)
            return pltpu.make_async_remote_copy(
                src_ref=out_ref.at[rows, :],
                dst_ref=out_ref.at[rows, :],
                send_sem=x_send.at[k, origin],
                recv_sem=x_recv.at[k, origin],
                device_id=(1 - my_x, my_y, my_z),
                device_id_type=pl.DeviceIdType.MESH,
            )

        def z_sends(k, t):
            @pl.when((my_z < N_Z - 1) & (my_z >= t))
            def _():
                z_rdma(k, my_z - t, 1, ze_send.at[k, t], ze_recv.at[k, t]).start()

            @pl.when((my_z > 0) & (my_z + t <= N_Z - 1))
            def _():
                z_rdma(k, my_z + t, -1, zw_send.at[k, t], zw_recv.at[k, t]).start()

        def z_waits(k, t):
            @pl.when(my_z >= t + 1)
            def _():
                z_rdma(k, my_z - 1 - t, -1, ze_send.at[k, t], ze_recv.at[k, t]).wait_recv()
                x_push(k, my_z - 1 - t).start()

            @pl.when(my_z <= N_Z - 2 - t)
            def _():
                z_rdma(k, my_z + 1 + t, 1, zw_send.at[k, t], zw_recv.at[k, t]).wait_recv()
                x_push(k, my_z + 1 + t).start()

        for s in range(N_Y - 1):
            p1_start(0, s)
            p1_wait(0, s)
        p1_finish(0)

        for k in range(K):
            nxt = k + 1
            for t in range(N_Z - 1):
                z_sends(k, t)
                if nxt < K:
                    if t > 0:
                        p1_wait(nxt, t - 1)
                    p1_start(nxt, t)
                z_waits(k, t)
            if nxt < K:
                p1_wait(nxt, N_Y - 2)
                p1_finish(nxt)

        for k in range(K):
            for o in range(N_Z):
                x_recv_desc(k, o).wait_recv()
                x_push(k, o).wait_send()
        for k in range(K):
            for s in range(N_Y - 1):
                hop_desc(k, s).wait_send()
            for t in range(N_Z - 1):
                @pl.when((my_z < N_Z - 1) & (my_z >= t))
                def _():
                    z_rdma(k, my_z - t, 1, ze_send.at[k, t], ze_recv.at[k, t]).wait_send()

                @pl.when((my_z > 0) & (my_z + t <= N_Z - 1))
                def _():
                    z_rdma(k, my_z + t, -1, zw_send.at[k, t], zw_recv.at[k, t]).wait_send()

    return pl.pallas_call(
        body,
        out_shape=jax.ShapeDtypeStruct((m, chunk), jnp.float32),
        in_specs=[pl.BlockSpec(memory_space=pl.ANY)],
        out_specs=pl.BlockSpec(memory_space=pltpu.MemorySpace.VMEM),
        scratch_shapes=[
            pltpu.VMEM((K, N_Y - 1, sub, chunk), jnp.float32),
            pltpu.VMEM((K, sub, chunk), jnp.float32),
            pltpu.SemaphoreType.DMA((K,)),
            pltpu.SemaphoreType.DMA((K, N_Y - 1)),
            pltpu.SemaphoreType.DMA((K, N_Y - 1)),
            pltpu.SemaphoreType.DMA((K, N_Z - 1)),
            pltpu.SemaphoreType.DMA((K, N_Z - 1)),
            pltpu.SemaphoreType.DMA((K, N_Z - 1)),
            pltpu.SemaphoreType.DMA((K, N_Z - 1)),
            pltpu.SemaphoreType.DMA((K, N_Z)),
            pltpu.SemaphoreType.DMA((K, N_Z)),
        ],
        compiler_params=pltpu.CompilerParams(
            collective_id=0,
            vmem_limit_bytes=63 * 1024 * 1024,
        ),
    )(x)
